# baseline (speedup 1.0000x reference)
import jax
import jax.numpy as jnp
from jax import lax
from jax.experimental import pallas as pl
from jax.experimental.pallas import tpu as pltpu
from jax.experimental.pallas import tpu_sc as plsc

N = 10000
E = 160000
D = 256
NC, NS = 2, 16
NW = NC * NS
SLICES = (64000, 96000)
CHUNK = 128
HALF = D // 2
ROWS_A = 1000
ROWS_C = 1000
GCH = 40
NSLOT = 3
ZCH = 200
NZ = N // ZCH


def _proj_body(q_ref, m_ref, w1a_ref, w1b_ref, b1_ref, q1_ref, k1_ref):
    q1_ref[...] = jnp.dot(q_ref[...], w1a_ref[...],
                          preferred_element_type=jnp.float32)
    k1_ref[...] = jnp.dot(m_ref[...], w1b_ref[...],
                          preferred_element_type=jnp.float32) + b1_ref[...]


def _gather_body_of(epw):
    NGC = epw // GCH

    def _gather_body(dst_h, src_h, q1_h, k1_h, mem_h, h1_h, kg_h,
                     dsti, srci, bufq, bufk, bufm,
                     gs0, gs1, gs2, wq0, wq1, wq2, wm0, wm1, wm2):
        w = lax.axis_index("s") * NC + lax.axis_index("c")
        gbase = pl.multiple_of(w * epw, epw)
        gsem = (gs0, gs1, gs2)
        wqsem = (wq0, wq1, wq2)
        wmsem = (wm0, wm1, wm2)

        pltpu.sync_copy(dst_h.at[pl.ds(gbase, epw)], dsti)
        pltpu.sync_copy(src_h.at[pl.ds(gbase, epw)], srci)

        def start(s, j, do_wait):
            if do_wait:
                pltpu.make_async_copy(bufq.at[s], h1_h.at[pl.ds(gbase, GCH)],
                                      wqsem[s]).wait()
                pltpu.make_async_copy(bufm.at[s], kg_h.at[pl.ds(gbase, GCH)],
                                      wmsem[s]).wait()
            off = pl.multiple_of(j * GCH, GCH)
            idq = dsti.at[pl.ds(off, GCH)]
            ids = srci.at[pl.ds(off, GCH)]
            pltpu.async_copy(q1_h.at[idq], bufq.at[s], gsem[s])
            pltpu.async_copy(k1_h.at[ids], bufk.at[s], gsem[s])
            pltpu.async_copy(mem_h.at[ids], bufm.at[s], gsem[s])

        def finish(s, j):
            for _ in range(3):
                pltpu.make_async_copy(q1_h.at[dsti.at[pl.ds(0, GCH)]],
                                      bufq.at[s], gsem[s]).wait()
            qv = bufq.at[s]
            kv = bufk.at[s]

            def arow(i, carry):
                for v in range(D // 16):
                    sl = pl.ds(v * 16, 16)
                    qv[i, sl] = qv[i, sl] + kv[i, sl]
                return carry

            lax.fori_loop(0, GCH, arow, 0)
            off = pl.multiple_of(j * GCH, GCH)
            pltpu.async_copy(bufq.at[s], h1_h.at[pl.ds(gbase + off, GCH)],
                             wqsem[s])
            pltpu.async_copy(bufm.at[s], kg_h.at[pl.ds(gbase + off, GCH)],
                             wmsem[s])

        for s in range(NSLOT):
            start(s, s, do_wait=False)

        def body(t, carry):
            for s in range(NSLOT):
                j = NSLOT * t + s

                @pl.when(j < NGC)
                def _(s=s, j=j):
                    finish(s, j)

            for s in range(NSLOT):
                nj = NSLOT * t + s + NSLOT

                @pl.when(nj < NGC)
                def _(s=s, nj=nj):
                    start(s, nj, do_wait=True)

            return carry

        lax.fori_loop(0, (NGC + NSLOT - 1) // NSLOT, body, 0)

        for s in range(NSLOT):
            pltpu.make_async_copy(bufq.at[s], h1_h.at[pl.ds(gbase, GCH)],
                                  wqsem[s]).wait()
            pltpu.make_async_copy(bufm.at[s], kg_h.at[pl.ds(gbase, GCH)],
                                  wmsem[s]).wait()

    return _gather_body


def _make_gather(e_sl):
    epw = e_sl // NW
    return pl.kernel(
        _gather_body_of(epw),
        out_type=[jax.ShapeDtypeStruct((e_sl, D), jnp.float32)] * 2,
        mesh=plsc.VectorSubcoreMesh(core_axis_name="c", subcore_axis_name="s",
                                    num_cores=NC, num_subcores=NS),
        scratch_types=[
            pltpu.VMEM((epw,), jnp.int32),
            pltpu.VMEM((epw,), jnp.int32),
            pltpu.VMEM((NSLOT, GCH, D), jnp.float32),
            pltpu.VMEM((NSLOT, GCH, D), jnp.float32),
            pltpu.VMEM((NSLOT, GCH, D), jnp.float32),
        ] + [pltpu.SemaphoreType.DMA] * 9,
    )


def _mlp_body(h1_ref, kg_ref, w2_ref, b2_ref, w3_ref, b3_ref,
              lng_ref, lnb_ref, out_ref):
    h1 = jnp.maximum(h1_ref[...], 0.0)
    h2 = jnp.dot(h1, w2_ref[...], preferred_element_type=jnp.float32)
    h2 = jnp.maximum(h2 + b2_ref[...], 0.0)
    h3 = jnp.dot(h2, w3_ref[...], preferred_element_type=jnp.float32)
    h3 = h3 + b3_ref[...]
    mu = jnp.mean(h3, axis=-1, keepdims=True)
    var = jnp.mean((h3 - mu) ** 2, axis=-1, keepdims=True)
    hn = (h3 - mu) * lax.rsqrt(var + 1e-5) * lng_ref[...] + lnb_ref[...]
    wgt = 1.0 / (1.0 + jnp.exp(-hn))
    out_ref[...] = kg_ref[...] * wgt


def _run_mlp(h1pre, kg, W2, b2r, W3, b3r, lngr, lnbr):
    e_sl = h1pre.shape[0]
    return pl.pallas_call(
        _mlp_body,
        grid=(e_sl // ROWS_C,),
        in_specs=[
            pl.BlockSpec((ROWS_C, D), lambda i: (i, 0)),
            pl.BlockSpec((ROWS_C, D), lambda i: (i, 0)),
            pl.BlockSpec((D, D), lambda i: (0, 0)),
            pl.BlockSpec((1, D), lambda i: (0, 0)),
            pl.BlockSpec((D, D), lambda i: (0, 0)),
            pl.BlockSpec((1, D), lambda i: (0, 0)),
            pl.BlockSpec((1, D), lambda i: (0, 0)),
            pl.BlockSpec((1, D), lambda i: (0, 0)),
        ],
        out_specs=pl.BlockSpec((ROWS_C, D), lambda i: (i, 0)),
        out_shape=jax.ShapeDtypeStruct((e_sl, D), jnp.float32),
    )(h1pre, kg, W2, b2r, W3, b3r, lngr, lnbr)


def _scatter_body_of(nchunks):
    def _scatter_body(dst_h, contrib_h, init_h, out_h, idxb, rows, acc,
                      ls0, ls1, ls2, ss0, ss1, ss2):
        c = lax.axis_index("c")
        s = lax.axis_index("s")
        lsem = (ls0, ls1, ls2)
        ssem = (ss0, ss1, ss2)

        def zbody(j, carry):
            ch = s + NS * j

            @pl.when(ch < NZ)
            def _():
                zbase = pl.multiple_of(ch * ZCH, ZCH)
                pltpu.sync_copy(
                    init_h.at[pl.ds(zbase, ZCH), pl.ds(c * HALF, HALF)],
                    acc.at[pl.ds(zbase, ZCH)])

            return carry

        lax.fori_loop(0, (NZ + NS - 1) // NS, zbody, 0)
        plsc.subcore_barrier()

        def start(sl, j, do_wait):
            if do_wait:
                pltpu.make_async_copy(rows.at[sl], acc.at[idxb.at[sl]],
                                      ssem[sl]).wait()
            ch = s + NS * j
            base = pl.multiple_of(ch * CHUNK, CHUNK)
            pltpu.async_copy(dst_h.at[pl.ds(base, CHUNK)], idxb.at[sl],
                             lsem[sl])
            pltpu.async_copy(
                contrib_h.at[pl.ds(base, CHUNK), pl.ds(c * HALF, HALF)],
                rows.at[sl], lsem[sl])

        def finish(sl):
            pltpu.make_async_copy(dst_h.at[pl.ds(0, CHUNK)], idxb.at[sl],
                                  lsem[sl]).wait()
            pltpu.make_async_copy(
                contrib_h.at[pl.ds(0, CHUNK), pl.ds(0, HALF)], rows.at[sl],
                lsem[sl]).wait()
            pltpu.async_copy(rows.at[sl], acc.at[idxb.at[sl]], ssem[sl],
                             add=True)

        NPW = (nchunks + NS - 1) // NS

        for sl in range(NSLOT):
            start(sl, sl, do_wait=False)

        def body(t, carry):
            for sl in range(NSLOT):
                j = NSLOT * t + sl

                @pl.when(s + NS * j < nchunks)
                def _(sl=sl, j=j):
                    finish(sl)

            for sl in range(NSLOT):
                nj = NSLOT * t + sl + NSLOT

                @pl.when(s + NS * nj < nchunks)
                def _(sl=sl, nj=nj):
                    start(sl, nj, do_wait=True)

            return carry

        lax.fori_loop(0, (NPW + NSLOT - 1) // NSLOT, body, 0)

        for sl in range(NSLOT):
            pltpu.make_async_copy(rows.at[sl], acc.at[idxb.at[sl]],
                                  ssem[sl]).wait()

        plsc.subcore_barrier()

        def obody(j, carry):
            ch = s + NS * j

            @pl.when(ch < NZ)
            def _():
                zbase = pl.multiple_of(ch * ZCH, ZCH)
                pltpu.sync_copy(acc.at[pl.ds(zbase, ZCH)],
                                out_h.at[pl.ds(zbase, ZCH),
                                         pl.ds(c * HALF, HALF)])

            return carry

        lax.fori_loop(0, (NZ + NS - 1) // NS, obody, 0)

    return _scatter_body


def _make_scatter(e_sl):
    return pl.kernel(
        _scatter_body_of(e_sl // CHUNK),
        out_type=jax.ShapeDtypeStruct((N, D), jnp.float32),
        mesh=plsc.VectorSubcoreMesh(core_axis_name="c", subcore_axis_name="s",
                                    num_cores=NC, num_subcores=NS),
        scratch_types=[
            pltpu.VMEM((NSLOT, CHUNK), jnp.int32),
            pltpu.VMEM((NSLOT, CHUNK, HALF), jnp.float32),
            pltpu.VMEM_SHARED((N, HALF), jnp.float32),
        ] + [pltpu.SemaphoreType.DMA] * 6,
    )


def kernel(query, memory, edge_index, W1, b1, W2, b2, W3, b3, ln_g, ln_b):
    dst = edge_index[0]
    src = edge_index[1]
    w1a = W1[:D]
    w1b = W1[D:]
    b1r = b1.reshape(1, D)
    b2r = b2.reshape(1, D)
    b3r = b3.reshape(1, D)
    lngr = ln_g.reshape(1, D)
    lnbr = ln_b.reshape(1, D)

    q1, k1 = pl.pallas_call(
        _proj_body,
        grid=(N // ROWS_A,),
        in_specs=[
            pl.BlockSpec((ROWS_A, D), lambda i: (i, 0)),
            pl.BlockSpec((ROWS_A, D), lambda i: (i, 0)),
            pl.BlockSpec((D, D), lambda i: (0, 0)),
            pl.BlockSpec((D, D), lambda i: (0, 0)),
            pl.BlockSpec((1, D), lambda i: (0, 0)),
        ],
        out_specs=[
            pl.BlockSpec((ROWS_A, D), lambda i: (i, 0)),
            pl.BlockSpec((ROWS_A, D), lambda i: (i, 0)),
        ],
        out_shape=[jax.ShapeDtypeStruct((N, D), jnp.float32)] * 2,
    )(query, memory, w1a, w1b, b1r)

    parts = []
    off = 0
    for e_sl in SLICES:
        d_sl = dst[off:off + e_sl]
        s_sl = src[off:off + e_sl]
        h1, kg = _make_gather(e_sl)(d_sl, s_sl, q1, k1, memory)
        parts.append((e_sl, d_sl, h1, kg))
        off += e_sl

    acc = jnp.zeros((N, D), jnp.float32)
    contribs = [(e_sl, d_sl, _run_mlp(h1, kg, W2, b2r, W3, b3r, lngr, lnbr))
                for e_sl, d_sl, h1, kg in parts]
    for e_sl, d_sl, contrib in contribs:
        acc = _make_scatter(e_sl)(d_sl, contrib, acc)
    return acc


# speedup vs baseline: 1.0770x; 1.0770x over previous
import jax
import jax.numpy as jnp
from jax import lax
from jax.experimental import pallas as pl
from jax.experimental.pallas import tpu as pltpu
from jax.experimental.pallas import tpu_sc as plsc

N = 10000
E = 160000
D = 256
NC, NS = 2, 16
NW = NC * NS
SLICES = (96000, 64000)
CHUNK = 128
HALF = D // 2
ROWS_A = 1000
ROWS_C = 2000
GCH = 40
NSLOT = 3
ZCH = 200
NZ = N // ZCH


def _proj_body(q_ref, m_ref, w1a_ref, w1b_ref, b1_ref, q1_ref, k1_ref):
    q1_ref[...] = jnp.dot(q_ref[...], w1a_ref[...],
                          preferred_element_type=jnp.float32)
    k1_ref[...] = jnp.dot(m_ref[...], w1b_ref[...],
                          preferred_element_type=jnp.float32) + b1_ref[...]


def _gather_body_of(epw):
    NGC = epw // GCH

    def _gather_body(dst_h, src_h, q1_h, k1_h, mem_h, h1_h, kg_h,
                     dsti, srci, bufq, bufk, bufm,
                     gs0, gs1, gs2, wq0, wq1, wq2, wm0, wm1, wm2):
        w = lax.axis_index("s") * NC + lax.axis_index("c")
        gbase = pl.multiple_of(w * epw, epw)
        gsem = (gs0, gs1, gs2)
        wqsem = (wq0, wq1, wq2)
        wmsem = (wm0, wm1, wm2)

        pltpu.sync_copy(dst_h.at[pl.ds(gbase, epw)], dsti)
        pltpu.sync_copy(src_h.at[pl.ds(gbase, epw)], srci)

        def start(s, j, do_wait):
            if do_wait:
                pltpu.make_async_copy(bufq.at[s], h1_h.at[pl.ds(gbase, GCH)],
                                      wqsem[s]).wait()
                pltpu.make_async_copy(bufm.at[s], kg_h.at[pl.ds(gbase, GCH)],
                                      wmsem[s]).wait()
            off = pl.multiple_of(j * GCH, GCH)
            idq = dsti.at[pl.ds(off, GCH)]
            ids = srci.at[pl.ds(off, GCH)]
            pltpu.async_copy(q1_h.at[idq], bufq.at[s], gsem[s])
            pltpu.async_copy(k1_h.at[ids], bufk.at[s], gsem[s])
            pltpu.async_copy(mem_h.at[ids], bufm.at[s], gsem[s])

        def finish(s, j):
            for _ in range(3):
                pltpu.make_async_copy(q1_h.at[dsti.at[pl.ds(0, GCH)]],
                                      bufq.at[s], gsem[s]).wait()
            qv = bufq.at[s]
            kv = bufk.at[s]

            def arow(i, carry):
                for v in range(D // 16):
                    sl = pl.ds(v * 16, 16)
                    qv[i, sl] = qv[i, sl] + kv[i, sl]
                return carry

            lax.fori_loop(0, GCH, arow, 0)
            off = pl.multiple_of(j * GCH, GCH)
            pltpu.async_copy(bufq.at[s], h1_h.at[pl.ds(gbase + off, GCH)],
                             wqsem[s])
            pltpu.async_copy(bufm.at[s], kg_h.at[pl.ds(gbase + off, GCH)],
                             wmsem[s])

        for s in range(NSLOT):
            start(s, s, do_wait=False)

        def body(t, carry):
            for s in range(NSLOT):
                j = NSLOT * t + s

                @pl.when(j < NGC)
                def _(s=s, j=j):
                    finish(s, j)

            for s in range(NSLOT):
                nj = NSLOT * t + s + NSLOT

                @pl.when(nj < NGC)
                def _(s=s, nj=nj):
                    start(s, nj, do_wait=True)

            return carry

        lax.fori_loop(0, (NGC + NSLOT - 1) // NSLOT, body, 0)

        for s in range(NSLOT):
            pltpu.make_async_copy(bufq.at[s], h1_h.at[pl.ds(gbase, GCH)],
                                  wqsem[s]).wait()
            pltpu.make_async_copy(bufm.at[s], kg_h.at[pl.ds(gbase, GCH)],
                                  wmsem[s]).wait()

    return _gather_body


def _make_gather(e_sl):
    epw = e_sl // NW
    return pl.kernel(
        _gather_body_of(epw),
        out_type=[jax.ShapeDtypeStruct((e_sl, D), jnp.float32)] * 2,
        mesh=plsc.VectorSubcoreMesh(core_axis_name="c", subcore_axis_name="s",
                                    num_cores=NC, num_subcores=NS),
        scratch_types=[
            pltpu.VMEM((epw,), jnp.int32),
            pltpu.VMEM((epw,), jnp.int32),
            pltpu.VMEM((NSLOT, GCH, D), jnp.float32),
            pltpu.VMEM((NSLOT, GCH, D), jnp.float32),
            pltpu.VMEM((NSLOT, GCH, D), jnp.float32),
        ] + [pltpu.SemaphoreType.DMA] * 9,
    )


def _mlp_body(h1_ref, kg_ref, w2_ref, b2_ref, w3_ref, b3_ref,
              lng_ref, lnb_ref, out_ref):
    h1 = jnp.maximum(h1_ref[...], 0.0)
    h2 = jnp.dot(h1, w2_ref[...], preferred_element_type=jnp.float32)
    h2 = jnp.maximum(h2 + b2_ref[...], 0.0)
    h3 = jnp.dot(h2, w3_ref[...], preferred_element_type=jnp.float32)
    h3 = h3 + b3_ref[...]
    mu = jnp.mean(h3, axis=-1, keepdims=True)
    var = jnp.mean((h3 - mu) ** 2, axis=-1, keepdims=True)
    hn = (h3 - mu) * lax.rsqrt(var + 1e-5) * lng_ref[...] + lnb_ref[...]
    wgt = 1.0 / (1.0 + jnp.exp(-hn))
    out_ref[...] = kg_ref[...] * wgt


def _run_mlp(h1pre, kg, W2, b2r, W3, b3r, lngr, lnbr):
    e_sl = h1pre.shape[0]
    return pl.pallas_call(
        _mlp_body,
        grid=(e_sl // ROWS_C,),
        in_specs=[
            pl.BlockSpec((ROWS_C, D), lambda i: (i, 0)),
            pl.BlockSpec((ROWS_C, D), lambda i: (i, 0)),
            pl.BlockSpec((D, D), lambda i: (0, 0)),
            pl.BlockSpec((1, D), lambda i: (0, 0)),
            pl.BlockSpec((D, D), lambda i: (0, 0)),
            pl.BlockSpec((1, D), lambda i: (0, 0)),
            pl.BlockSpec((1, D), lambda i: (0, 0)),
            pl.BlockSpec((1, D), lambda i: (0, 0)),
        ],
        out_specs=pl.BlockSpec((ROWS_C, D), lambda i: (i, 0)),
        out_shape=jax.ShapeDtypeStruct((e_sl, D), jnp.float32),
    )(h1pre, kg, W2, b2r, W3, b3r, lngr, lnbr)


def _scatter_body_of(nchunks):
    def _scatter_body(dst_h, contrib_h, init_h, out_h, idxb, rows, acc,
                      ls0, ls1, ls2, ss0, ss1, ss2):
        c = lax.axis_index("c")
        s = lax.axis_index("s")
        lsem = (ls0, ls1, ls2)
        ssem = (ss0, ss1, ss2)

        def zbody(j, carry):
            ch = s + NS * j

            @pl.when(ch < NZ)
            def _():
                zbase = pl.multiple_of(ch * ZCH, ZCH)
                pltpu.sync_copy(
                    init_h.at[pl.ds(zbase, ZCH), pl.ds(c * HALF, HALF)],
                    acc.at[pl.ds(zbase, ZCH)])

            return carry

        lax.fori_loop(0, (NZ + NS - 1) // NS, zbody, 0)
        plsc.subcore_barrier()

        def start(sl, j, do_wait):
            if do_wait:
                pltpu.make_async_copy(rows.at[sl], acc.at[idxb.at[sl]],
                                      ssem[sl]).wait()
            ch = s + NS * j
            base = pl.multiple_of(ch * CHUNK, CHUNK)
            pltpu.async_copy(dst_h.at[pl.ds(base, CHUNK)], idxb.at[sl],
                             lsem[sl])
            pltpu.async_copy(
                contrib_h.at[pl.ds(base, CHUNK), pl.ds(c * HALF, HALF)],
                rows.at[sl], lsem[sl])

        def finish(sl):
            pltpu.make_async_copy(dst_h.at[pl.ds(0, CHUNK)], idxb.at[sl],
                                  lsem[sl]).wait()
            pltpu.make_async_copy(
                contrib_h.at[pl.ds(0, CHUNK), pl.ds(0, HALF)], rows.at[sl],
                lsem[sl]).wait()
            pltpu.async_copy(rows.at[sl], acc.at[idxb.at[sl]], ssem[sl],
                             add=True)

        NPW = (nchunks + NS - 1) // NS

        for sl in range(NSLOT):
            start(sl, sl, do_wait=False)

        def body(t, carry):
            for sl in range(NSLOT):
                j = NSLOT * t + sl

                @pl.when(s + NS * j < nchunks)
                def _(sl=sl, j=j):
                    finish(sl)

            for sl in range(NSLOT):
                nj = NSLOT * t + sl + NSLOT

                @pl.when(s + NS * nj < nchunks)
                def _(sl=sl, nj=nj):
                    start(sl, nj, do_wait=True)

            return carry

        lax.fori_loop(0, (NPW + NSLOT - 1) // NSLOT, body, 0)

        for sl in range(NSLOT):
            pltpu.make_async_copy(rows.at[sl], acc.at[idxb.at[sl]],
                                  ssem[sl]).wait()

        plsc.subcore_barrier()

        def obody(j, carry):
            ch = s + NS * j

            @pl.when(ch < NZ)
            def _():
                zbase = pl.multiple_of(ch * ZCH, ZCH)
                pltpu.sync_copy(acc.at[pl.ds(zbase, ZCH)],
                                out_h.at[pl.ds(zbase, ZCH),
                                         pl.ds(c * HALF, HALF)])

            return carry

        lax.fori_loop(0, (NZ + NS - 1) // NS, obody, 0)

    return _scatter_body


def _make_scatter(e_sl):
    return pl.kernel(
        _scatter_body_of(e_sl // CHUNK),
        out_type=jax.ShapeDtypeStruct((N, D), jnp.float32),
        mesh=plsc.VectorSubcoreMesh(core_axis_name="c", subcore_axis_name="s",
                                    num_cores=NC, num_subcores=NS),
        scratch_types=[
            pltpu.VMEM((NSLOT, CHUNK), jnp.int32),
            pltpu.VMEM((NSLOT, CHUNK, HALF), jnp.float32),
            pltpu.VMEM_SHARED((N, HALF), jnp.float32),
        ] + [pltpu.SemaphoreType.DMA] * 6,
    )


def kernel(query, memory, edge_index, W1, b1, W2, b2, W3, b3, ln_g, ln_b):
    dst = edge_index[0]
    src = edge_index[1]
    w1a = W1[:D]
    w1b = W1[D:]
    b1r = b1.reshape(1, D)
    b2r = b2.reshape(1, D)
    b3r = b3.reshape(1, D)
    lngr = ln_g.reshape(1, D)
    lnbr = ln_b.reshape(1, D)

    q1, k1 = pl.pallas_call(
        _proj_body,
        grid=(N // ROWS_A,),
        in_specs=[
            pl.BlockSpec((ROWS_A, D), lambda i: (i, 0)),
            pl.BlockSpec((ROWS_A, D), lambda i: (i, 0)),
            pl.BlockSpec((D, D), lambda i: (0, 0)),
            pl.BlockSpec((D, D), lambda i: (0, 0)),
            pl.BlockSpec((1, D), lambda i: (0, 0)),
        ],
        out_specs=[
            pl.BlockSpec((ROWS_A, D), lambda i: (i, 0)),
            pl.BlockSpec((ROWS_A, D), lambda i: (i, 0)),
        ],
        out_shape=[jax.ShapeDtypeStruct((N, D), jnp.float32)] * 2,
    )(query, memory, w1a, w1b, b1r)

    parts = []
    off = 0
    for e_sl in SLICES:
        d_sl = dst[off:off + e_sl]
        s_sl = src[off:off + e_sl]
        h1, kg = _make_gather(e_sl)(d_sl, s_sl, q1, k1, memory)
        parts.append((e_sl, d_sl, h1, kg))
        off += e_sl

    acc = jnp.zeros((N, D), jnp.float32)
    contribs = [(e_sl, d_sl, _run_mlp(h1, kg, W2, b2r, W3, b3r, lngr, lnbr))
                for e_sl, d_sl, h1, kg in parts]
    for e_sl, d_sl, contrib in contribs:
        acc = _make_scatter(e_sl)(d_sl, contrib, acc)
    return acc
